# Initial kernel scaffold; baseline (speedup 1.0000x reference)
import jax
import jax.numpy as jnp
from jax.experimental import pallas as pl


def kernel(item_seq_batch, table):
    raise NotImplementedError("write your pallas kernel here")


# speedup vs baseline: 4.6795x; 4.6795x over previous
import functools

import jax
import jax.numpy as jnp
from jax import lax
from jax.experimental import pallas as pl
from jax.experimental.pallas import tpu as pltpu
from jax.experimental.pallas import tpu_sc as plsc

_NUM_EMB = 100000
_DIM = 64
_B = 4096
_L = 50
_BL = _B * _L
_NW = 32
_NPER = _BL // _NW
_G = 128
_NG = _NPER // _G
_CHG = 5
_CH = _CHG * _G
_NCH = _NG // _CHG


def _build():
    mesh = plsc.VectorSubcoreMesh(core_axis_name="c", subcore_axis_name="s")

    @functools.partial(
        pl.kernel,
        mesh=mesh,
        out_type=jax.ShapeDtypeStruct((_BL, _DIM), jnp.float32),
        scratch_types=[
            pltpu.VMEM((_NG, _G), jnp.int32),
            pltpu.VMEM((2, _CH, _DIM), jnp.float32),
            pltpu.SemaphoreType.DMA,
            pltpu.SemaphoreType.DMA,
        ],
        compiler_params=pltpu.CompilerParams(use_tc_tiling_on_sc=False),
    )
    def gather_kernel(idx_hbm, table_hbm, out_hbm, idx_v, buf_v, sem0, sem1):
        wid = lax.axis_index("s") * 2 + lax.axis_index("c")
        base = wid * _NPER
        pltpu.sync_copy(idx_hbm.at[wid], idx_v)

        sems = (sem0, sem1)

        def fire(c, b):
            for j in range(_CHG):
                pltpu.async_copy(
                    table_hbm.at[idx_v.at[c * _CHG + j]],
                    buf_v.at[b].at[pl.ds(j * _G, _G)],
                    sems[b],
                )

        def drain(c, b):
            for j in range(_CHG):
                pltpu.make_async_copy(
                    table_hbm.at[idx_v.at[c * _CHG + j]],
                    buf_v.at[b].at[pl.ds(j * _G, _G)],
                    sems[b],
                ).wait()

        def scat(c, b):
            pltpu.sync_copy(buf_v.at[b], out_hbm.at[pl.ds(base + c * _CH, _CH)])

        fire(0, 0)
        def body(i, carry):
            c0 = i * 2
            for d in range(2):
                c = c0 + d
                fire(c + 1, 1 - d)
                drain(c, d)
                scat(c, d)
            return carry
        lax.fori_loop(0, (_NCH - 2) // 2, body, 0)
        c = _NCH - 2
        fire(c + 1, 1)
        drain(c, 0)
        scat(c, 0)
        drain(c + 1, 1)
        scat(c + 1, 1)

    return gather_kernel


_gather_cache = []


def kernel(item_seq_batch, table):
    if not _gather_cache:
        _gather_cache.append(_build())
    idx = item_seq_batch.astype(jnp.int32).reshape(_NW, _NG, _G)
    out = _gather_cache[0](idx, table)
    return out.reshape(_B, _L, _DIM)


# speedup vs baseline: 4.6818x; 1.0005x over previous
import functools

import jax
import jax.numpy as jnp
from jax import lax
from jax.experimental import pallas as pl
from jax.experimental.pallas import tpu as pltpu
from jax.experimental.pallas import tpu_sc as plsc

_NUM_EMB = 100000
_DIM = 64
_B = 4096
_L = 50
_BL = _B * _L
_NW = 32
_NPER = _BL // _NW
_G = 128
_NG = _NPER // _G
_CHG = 2
_CH = _CHG * _G
_NCH = _NG // _CHG
_R = 5


def _build():
    mesh = plsc.VectorSubcoreMesh(core_axis_name="c", subcore_axis_name="s")

    @functools.partial(
        pl.kernel,
        mesh=mesh,
        out_type=jax.ShapeDtypeStruct((_BL, _DIM), jnp.float32),
        scratch_types=[
            pltpu.VMEM((_NG, _G), jnp.int32),
            pltpu.VMEM((_R, _CH, _DIM), jnp.float32),
            [pltpu.SemaphoreType.DMA] * _R,
            [pltpu.SemaphoreType.DMA] * _R,
        ],
        compiler_params=pltpu.CompilerParams(use_tc_tiling_on_sc=False),
    )
    def gather_kernel(idx_hbm, table_hbm, out_hbm, idx_v, buf_v, gsems, ssems):
        wid = lax.axis_index("s") * 2 + lax.axis_index("c")
        base = wid * _NPER
        pltpu.sync_copy(idx_hbm.at[wid], idx_v)

        def fire_g(c, b):
            for j in range(_CHG):
                pltpu.async_copy(
                    table_hbm.at[idx_v.at[c * _CHG + j]],
                    buf_v.at[b].at[pl.ds(j * _G, _G)],
                    gsems[b],
                )

        def drain_g(c, b):
            for j in range(_CHG):
                pltpu.make_async_copy(
                    table_hbm.at[idx_v.at[c * _CHG + j]],
                    buf_v.at[b].at[pl.ds(j * _G, _G)],
                    gsems[b],
                ).wait()

        def fire_s(c, b):
            pltpu.async_copy(
                buf_v.at[b], out_hbm.at[pl.ds(base + c * _CH, _CH)], ssems[b]
            )

        def drain_s(b):
            pltpu.make_async_copy(
                buf_v.at[b], out_hbm.at[pl.ds(base, _CH)], ssems[b]
            ).wait()

        for c in range(_R - 1):
            fire_g(c, c)

        def body(i, carry):
            for d in range(_R):
                c = i * _R + d
                drain_g(c, d)
                fire_s(c, d)
                g = c + _R - 1
                bg = (d + _R - 1) % _R

                @pl.when(g < _NCH)
                def _():
                    @pl.when(c >= 1)
                    def _():
                        drain_s(bg)

                    fire_g(g, bg)

            return carry

        lax.fori_loop(0, _NCH // _R, body, 0)
        for b in range(_R):
            drain_s(b)

    return gather_kernel


_gather_cache = []


def kernel(item_seq_batch, table):
    if not _gather_cache:
        _gather_cache.append(_build())
    idx = item_seq_batch.astype(jnp.int32).reshape(_NW, _NG, _G)
    out = _gather_cache[0](idx, table)
    return out.reshape(_B, _L, _DIM)


# speedup vs baseline: 4.6883x; 1.0014x over previous
import functools

import jax
import jax.numpy as jnp
from jax import lax
from jax.experimental import pallas as pl
from jax.experimental.pallas import tpu as pltpu
from jax.experimental.pallas import tpu_sc as plsc

_NUM_EMB = 100000
_DIM = 64
_B = 4096
_L = 50
_NW = 32
_SPW = _B // _NW
_CB = 8
_NCB = _SPW // _CB
_R = 4


def _build():
    mesh = plsc.VectorSubcoreMesh(core_axis_name="c", subcore_axis_name="s")

    @functools.partial(
        pl.kernel,
        mesh=mesh,
        out_type=jax.ShapeDtypeStruct((_B, _L, _DIM), jnp.float32),
        scratch_types=[
            pltpu.VMEM((_SPW, _L), jnp.int32),
            pltpu.VMEM((_R, _CB, _L, _DIM), jnp.float32),
            [pltpu.SemaphoreType.DMA] * _R,
            [pltpu.SemaphoreType.DMA] * _R,
        ],
        compiler_params=pltpu.CompilerParams(use_tc_tiling_on_sc=False),
    )
    def gather_kernel(idx_hbm, table_hbm, out_hbm, idx_v, buf_v, gsems, ssems):
        wid = lax.axis_index("s") * 2 + lax.axis_index("c")
        base = wid * _SPW
        pltpu.sync_copy(idx_hbm.at[pl.ds(base, _SPW)], idx_v)

        def fire_g(c, b):
            for j in range(_CB):
                pltpu.async_copy(
                    table_hbm.at[idx_v.at[c * _CB + j]],
                    buf_v.at[b].at[j],
                    gsems[b],
                )

        def drain_g(c, b):
            for j in range(_CB):
                pltpu.make_async_copy(
                    table_hbm.at[idx_v.at[c * _CB + j]],
                    buf_v.at[b].at[j],
                    gsems[b],
                ).wait()

        def fire_s(c, b):
            pltpu.async_copy(
                buf_v.at[b], out_hbm.at[pl.ds(base + c * _CB, _CB)], ssems[b]
            )

        def drain_s(b):
            pltpu.make_async_copy(
                buf_v.at[b], out_hbm.at[pl.ds(base, _CB)], ssems[b]
            ).wait()

        for c in range(_R - 1):
            fire_g(c, c)

        def body(i, carry):
            for d in range(_R):
                c = i * _R + d
                drain_g(c, d)
                fire_s(c, d)
                g = c + _R - 1
                bg = (d + _R - 1) % _R

                @pl.when(g < _NCB)
                def _():
                    @pl.when(c >= 1)
                    def _():
                        drain_s(bg)

                    fire_g(g, bg)

            return carry

        lax.fori_loop(0, _NCB // _R, body, 0)
        for b in range(_R):
            drain_s(b)

    return gather_kernel


_gather_cache = []


def kernel(item_seq_batch, table):
    if not _gather_cache:
        _gather_cache.append(_build())
    idx = item_seq_batch.astype(jnp.int32)
    return _gather_cache[0](idx, table)
